# baseline (speedup 1.0000x reference)
import functools

import jax
import jax.numpy as jnp
from jax import lax
from jax.experimental import pallas as pl
from jax.experimental.pallas import tpu as pltpu
from jax.experimental.pallas import tpu_sc as plsc

_B, _S, _D = 4, 4096, 1024
_NC, _NS, _L = 2, 16, 16
_NW = _NC * _NS
_ROWS_PER_W = _S // _NW
_R = 8
_CHUNK = _R * _D
_N_CHUNKS = _ROWS_PER_W // _R


def _sc_body(x_hbm, pe_hbm, out_hbm, pe_v, x_v, o_v):
    wid = lax.axis_index("s") * _NC + lax.axis_index("c")
    base0 = wid * (_ROWS_PER_W * _D)

    def chunk_body(c, carry):
        pe_base = base0 + c * _CHUNK
        pltpu.sync_copy(pe_hbm.at[pl.ds(pe_base, _CHUNK)], pe_v)
        for b in range(_B):
            x_base = b * (_S * _D) + pe_base
            pltpu.sync_copy(x_hbm.at[pl.ds(x_base, _CHUNK)], x_v)

            def add_body(i, acc):
                sl = pl.ds(i * _L, _L)
                o_v[sl] = x_v[sl] + pe_v[sl]
                return acc

            lax.fori_loop(0, _CHUNK // _L, add_body, 0, unroll=8)
            pltpu.sync_copy(o_v, out_hbm.at[pl.ds(x_base, _CHUNK)])
        return carry

    lax.fori_loop(0, _N_CHUNKS, chunk_body, 0)


@functools.partial(jax.jit, static_argnames=())
def _sc_call(x_flat, pe_flat):
    mesh = plsc.VectorSubcoreMesh(core_axis_name="c", subcore_axis_name="s")
    k = functools.partial(
        pl.kernel,
        mesh=mesh,
        out_type=jax.ShapeDtypeStruct((_B * _S * _D,), jnp.float32),
        scratch_types=[
            pltpu.VMEM((_CHUNK,), jnp.float32),
            pltpu.VMEM((_CHUNK,), jnp.float32),
            pltpu.VMEM((_CHUNK,), jnp.float32),
        ],
    )(_sc_body)
    return k(x_flat, pe_flat)


def kernel(x, pos_emb):
    B, S, D = x.shape
    pe = pos_emb[:S]
    out_flat = _sc_call(x.reshape(-1), pe.reshape(-1))
    return out_flat.reshape(B, S, D)


# speedup vs baseline: 1.2956x; 1.2956x over previous
import functools

import jax
import jax.numpy as jnp
from jax import lax
from jax.experimental import pallas as pl
from jax.experimental.pallas import tpu as pltpu
from jax.experimental.pallas import tpu_sc as plsc

_B, _S, _D = 4, 4096, 1024
_NC, _NS, _L = 2, 16, 16
_NW = _NC * _NS
_ROWS_PER_W = _S // _NW
_R = 16
_CHUNK = _R * _D
_N_CHUNKS = _ROWS_PER_W // _R
_UNITS = _N_CHUNKS * _B


def _sc_body(x_hbm, pe_hbm, out_hbm,
             pe_v0, pe_v1, x_v0, x_v1, o_v0, o_v1,
             pe_s0, pe_s1, x_s0, x_s1, o_s0, o_s1):
    pe_bufs = (pe_v0, pe_v1)
    x_bufs = (x_v0, x_v1)
    o_bufs = (o_v0, o_v1)
    pe_sems = (pe_s0, pe_s1)
    x_sems = (x_s0, x_s1)
    o_sems = (o_s0, o_s1)

    wid = lax.axis_index("s") * _NC + lax.axis_index("c")
    base0 = wid * (_ROWS_PER_W * _D)

    def start_pe(c):
        return pltpu.async_copy(
            pe_hbm.at[pl.ds(base0 + c * _CHUNK, _CHUNK)],
            pe_bufs[c % 2], pe_sems[c % 2])

    def start_x(u):
        c, b = divmod(u, _B)
        src = x_hbm.at[pl.ds(b * (_S * _D) + base0 + c * _CHUNK, _CHUNK)]
        return pltpu.async_copy(src, x_bufs[u % 2], x_sems[u % 2])

    def start_out(u):
        c, b = divmod(u, _B)
        dst = out_hbm.at[pl.ds(b * (_S * _D) + base0 + c * _CHUNK, _CHUNK)]
        return pltpu.async_copy(o_bufs[u % 2], dst, o_sems[u % 2])

    pe_dma = [None] * (_N_CHUNKS + 1)
    x_dma = [None] * (_UNITS + 1)
    o_dma = [None] * (_UNITS + 1)

    pe_dma[0] = start_pe(0)
    x_dma[0] = start_x(0)

    for u in range(_UNITS):
        c, b = divmod(u, _B)
        if u + 1 < _UNITS:
            x_dma[u + 1] = start_x(u + 1)
        if b == 0 and c + 1 < _N_CHUNKS:
            pe_dma[c + 1] = start_pe(c + 1)

        x_dma[u].wait()
        if b == 0:
            pe_dma[c].wait()
        if u >= 2:
            o_dma[u - 2].wait()

        x_v = x_bufs[u % 2]
        pe_v = pe_bufs[c % 2]
        o_v = o_bufs[u % 2]

        def add_body(i, acc, x_v=x_v, pe_v=pe_v, o_v=o_v):
            sl = pl.ds(i * _L, _L)
            o_v[sl] = x_v[sl] + pe_v[sl]
            return acc

        lax.fori_loop(0, _CHUNK // _L, add_body, 0, unroll=8)
        o_dma[u] = start_out(u)

    o_dma[_UNITS - 2].wait()
    o_dma[_UNITS - 1].wait()


@jax.jit
def _sc_call(x_flat, pe_flat):
    mesh = plsc.VectorSubcoreMesh(core_axis_name="c", subcore_axis_name="s")
    k = functools.partial(
        pl.kernel,
        mesh=mesh,
        out_type=jax.ShapeDtypeStruct((_B * _S * _D,), jnp.float32),
        scratch_types=[
            pltpu.VMEM((_CHUNK,), jnp.float32),
            pltpu.VMEM((_CHUNK,), jnp.float32),
            pltpu.VMEM((_CHUNK,), jnp.float32),
            pltpu.VMEM((_CHUNK,), jnp.float32),
            pltpu.VMEM((_CHUNK,), jnp.float32),
            pltpu.VMEM((_CHUNK,), jnp.float32),
            pltpu.SemaphoreType.DMA,
            pltpu.SemaphoreType.DMA,
            pltpu.SemaphoreType.DMA,
            pltpu.SemaphoreType.DMA,
            pltpu.SemaphoreType.DMA,
            pltpu.SemaphoreType.DMA,
        ],
    )(_sc_body)
    return k(x_flat, pe_flat)


def kernel(x, pos_emb):
    B, S, D = x.shape
    pe = pos_emb[:S]
    out_flat = _sc_call(x.reshape(-1), pe.reshape(-1))
    return out_flat.reshape(B, S, D)


# speedup vs baseline: 2.0139x; 1.5544x over previous
import functools

import jax
import jax.numpy as jnp
from jax import lax
from jax.experimental import pallas as pl
from jax.experimental.pallas import tpu as pltpu
from jax.experimental.pallas import tpu_sc as plsc

_B, _S, _D = 4, 4096, 1024
_NC, _NS, _L = 2, 16, 16
_NW = _NC * _NS
_ROWS_PER_W = _S // _NW
_R = 16
_CHUNK = _R * _D
_N_CHUNKS = _ROWS_PER_W // _R
_UNITS = _N_CHUNKS * _B


def _sc_body(x_hbm, pe_hbm, out_hbm,
             pe_v0, pe_v1, x_v0, x_v1, o_v0, o_v1,
             pe_s0, pe_s1, x_s0, x_s1, o_s0, o_s1):
    pe_bufs = (pe_v0, pe_v1)
    x_bufs = (x_v0, x_v1)
    o_bufs = (o_v0, o_v1)
    pe_sems = (pe_s0, pe_s1)
    x_sems = (x_s0, x_s1)
    o_sems = (o_s0, o_s1)

    wid = lax.axis_index("s") * _NC + lax.axis_index("c")
    base0 = wid * (_ROWS_PER_W * _D)

    def start_pe(c):
        return pltpu.async_copy(
            pe_hbm.at[pl.ds(base0 + c * _CHUNK, _CHUNK)],
            pe_bufs[c % 2], pe_sems[c % 2])

    def start_x(u):
        c, b = divmod(u, _B)
        src = x_hbm.at[pl.ds(b * (_S * _D) + base0 + c * _CHUNK, _CHUNK)]
        return pltpu.async_copy(src, x_bufs[u % 2], x_sems[u % 2])

    def start_out(u):
        c, b = divmod(u, _B)
        dst = out_hbm.at[pl.ds(b * (_S * _D) + base0 + c * _CHUNK, _CHUNK)]
        return pltpu.async_copy(o_bufs[u % 2], dst, o_sems[u % 2])

    pe_dma = [None] * (_N_CHUNKS + 1)
    x_dma = [None] * (_UNITS + 1)
    o_dma = [None] * (_UNITS + 1)

    pe_dma[0] = start_pe(0)
    x_dma[0] = start_x(0)

    for u in range(_UNITS):
        c, b = divmod(u, _B)
        if u + 1 < _UNITS:
            x_dma[u + 1] = start_x(u + 1)
        if b == 0 and c + 1 < _N_CHUNKS:
            pe_dma[c + 1] = start_pe(c + 1)

        x_dma[u].wait()
        if b == 0:
            pe_dma[c].wait()
        if u >= 2:
            o_dma[u - 2].wait()

        x_v = x_bufs[u % 2]
        pe_v = pe_bufs[c % 2]
        o_v = o_bufs[u % 2]

        @plsc.parallel_loop(0, _CHUNK, step=_L, unroll=8)
        def add_body(i, x_v=x_v, pe_v=pe_v, o_v=o_v):
            sl = pl.ds(i, _L)
            o_v[sl] = x_v[sl] + pe_v[sl]

        o_dma[u] = start_out(u)

    o_dma[_UNITS - 2].wait()
    o_dma[_UNITS - 1].wait()


@jax.jit
def _sc_call(x_flat, pe_flat):
    mesh = plsc.VectorSubcoreMesh(core_axis_name="c", subcore_axis_name="s")
    k = functools.partial(
        pl.kernel,
        mesh=mesh,
        out_type=jax.ShapeDtypeStruct((_B * _S * _D,), jnp.float32),
        scratch_types=[
            pltpu.VMEM((_CHUNK,), jnp.float32),
            pltpu.VMEM((_CHUNK,), jnp.float32),
            pltpu.VMEM((_CHUNK,), jnp.float32),
            pltpu.VMEM((_CHUNK,), jnp.float32),
            pltpu.VMEM((_CHUNK,), jnp.float32),
            pltpu.VMEM((_CHUNK,), jnp.float32),
            pltpu.SemaphoreType.DMA,
            pltpu.SemaphoreType.DMA,
            pltpu.SemaphoreType.DMA,
            pltpu.SemaphoreType.DMA,
            pltpu.SemaphoreType.DMA,
            pltpu.SemaphoreType.DMA,
        ],
    )(_sc_body)
    return k(x_flat, pe_flat)


def kernel(x, pos_emb):
    B, S, D = x.shape
    pe = pos_emb[:S]
    out_flat = _sc_call(x.reshape(-1), pe.reshape(-1))
    return out_flat.reshape(B, S, D)
